# baseline (speedup 1.0000x reference)
import functools

import jax
import jax.numpy as jnp
from jax import lax
from jax.experimental import pallas as pl
from jax.experimental.pallas import tpu as pltpu
from jax.experimental.pallas import tpu_sc as plsc

_ROWS = 2048
_COLS = 2048
_TOT = _ROWS * _COLS

_NC = 2
_NS = 16
_NW = _NC * _NS
_PER_W = _TOT // _NW
_CH = 16384
_NCH = _PER_W // _CH

_GROUPS = (
    ((0, 4, 8, 12), (2, 6, 10, 14), 18),
    ((1, 5, 9, 13), (3, 7, 11, 15), 19),
    ((2, 6, 10, 14), (4, 8, 12, 16), None),
    ((3, 7, 11, 15), (5, 9, 13, 17), None),
)


def _sc_body(wb, *refs):
    p = refs[0:16]
    o = refs[16:22]
    wv, b0, b1, b2, b3, ob, sb, sem = refs[22:]
    bufs = (b0, b1, b2, b3)
    wid = lax.axis_index("s") * _NC + lax.axis_index("c")
    base = wid * _PER_W
    pltpu.sync_copy(wb, wv)
    for g, (pidx, widx, sidx) in enumerate(_GROUPS):
        w0, w1, w2, w3 = (wv[i] for i in widx)
        ws = wv[sidx] if sidx is not None else None
        out_ref = o[g]
        scaled_ref = o[4 + g] if sidx is not None else None

        def chunk_body(c, _, pidx=pidx, w0=w0, w1=w1, w2=w2, w3=w3, ws=ws,
                       out_ref=out_ref, scaled_ref=scaled_ref):
            off = base + c * _CH
            cps = [pltpu.async_copy(p[pidx[k]].at[pl.ds(off, _CH)], bufs[k], sem)
                   for k in range(4)]
            for cp in cps:
                cp.wait()

            def vec_body(i, _):
                ds = pl.ds(i * 16, 16)
                s = b0[ds] * w0 + b1[ds] * w1 + b2[ds] * w2 + b3[ds] * w3
                ob[ds] = s
                if ws is not None:
                    sb[ds] = s * ws
                return 0

            lax.fori_loop(0, _CH // 16, vec_body, 0, unroll=8)
            pltpu.sync_copy(ob, out_ref.at[pl.ds(off, _CH)])
            if scaled_ref is not None:
                pltpu.sync_copy(sb, scaled_ref.at[pl.ds(off, _CH)])
            return 0

        lax.fori_loop(0, _NCH, chunk_body, 0)


def _sc_call(weights, params_flat):
    wb = jnp.broadcast_to(weights.reshape(20, 1), (20, 16))
    mesh = plsc.VectorSubcoreMesh(core_axis_name="c", subcore_axis_name="s",
                                  num_cores=_NC, num_subcores=_NS)
    f = pl.kernel(
        _sc_body,
        out_type=[jax.ShapeDtypeStruct((_TOT,), jnp.float32)] * 6,
        mesh=mesh,
        scratch_types=[
            pltpu.VMEM((20, 16), jnp.float32),
            pltpu.VMEM((_CH,), jnp.float32),
            pltpu.VMEM((_CH,), jnp.float32),
            pltpu.VMEM((_CH,), jnp.float32),
            pltpu.VMEM((_CH,), jnp.float32),
            pltpu.VMEM((_CH,), jnp.float32),
            pltpu.VMEM((_CH,), jnp.float32),
            pltpu.SemaphoreType.DMA,
        ],
    )
    return f(wb, *params_flat)


def kernel(loss, prev_loss, weights, param_0, param_1, param_2, param_3,
           param_4, param_5, param_6, param_7, param_8, param_9, param_10,
           param_11, param_12, param_13, param_14, param_15):
    del loss, prev_loss
    params = (param_0, param_1, param_2, param_3, param_4, param_5, param_6,
              param_7, param_8, param_9, param_10, param_11, param_12,
              param_13, param_14, param_15)
    outs = _sc_call(weights, tuple(p.reshape(_TOT) for p in params))
    return tuple(x.reshape(_ROWS, _COLS) for x in outs)


# speedup vs baseline: 5.3573x; 5.3573x over previous
import jax
import jax.numpy as jnp
from jax import lax
from jax.experimental import pallas as pl
from jax.experimental.pallas import tpu as pltpu
from jax.experimental.pallas import tpu_sc as plsc

_ROWS = 2048
_COLS = 2048

_NC = 2
_NS = 16
_NW = _NC * _NS

_TR_PER_W = (_ROWS // 8) // _NW
_CW = _COLS // 2

_GROUPS = (
    ((0, 4, 8, 12), (2, 6, 10, 14), 18),
    ((1, 5, 9, 13), (3, 7, 11, 15), 19),
    ((2, 6, 10, 14), (4, 8, 12, 16), None),
    ((3, 7, 11, 15), (5, 9, 13, 17), None),
)


def _compute_chunk(bufs, ob, sb, w0, w1, w2, w3, ws):
    @plsc.parallel_loop(0, 8 * (_CW // 16), unroll=8)
    def _(t):
        i = t >> 6
        ds = pl.ds((t & 63) * 16, 16)
        b0, b1, b2, b3 = bufs
        s = (b0[i, ds] * w0 + b1[i, ds] * w1) + (b2[i, ds] * w2 + b3[i, ds] * w3)
        ob[i, ds] = s
        if ws is not None:
            sb[i, ds] = s * ws


def _sc_body(wb, *refs):
    p = refs[0:16]
    o = refs[16:22]
    (wv, b00, b01, b02, b03, b10, b11, b12, b13, ob0, ob1, sb0, sb1,
     sem_a, sem_b, sem_oa, sem_ob) = refs[22:]
    bufs0 = (b00, b01, b02, b03)
    bufs1 = (b10, b11, b12, b13)
    wid = lax.axis_index("s") * _NC + lax.axis_index("c")
    base_tr = wid * _TR_PER_W
    pltpu.sync_copy(wb, wv)

    for g, (pidx, widx, sidx) in enumerate(_GROUPS):
        w0, w1, w2, w3 = (wv[i] for i in widx)
        ws = wv[sidx] if sidx is not None else None
        out_ref = o[g]
        scaled_ref = o[4 + g] if sidx is not None else None

        def in_slice(k, tr, h, pidx=pidx):
            return p[pidx[k]].at[pl.ds(tr * 8, 8), pl.ds(h * _CW, _CW)]

        def issue_in(tr, h, bufs, sem, pidx=pidx):
            for k in range(4):
                pltpu.async_copy(in_slice(k, tr, h, pidx), bufs[k], sem)

        def wait_in(tr, h, bufs, sem, pidx=pidx):
            for k in range(4):
                pltpu.make_async_copy(in_slice(k, tr, h, pidx), bufs[k], sem).wait()

        def issue_out(tr, h, obuf, sbuf, sem,
                      out_ref=out_ref, scaled_ref=scaled_ref):
            pltpu.async_copy(obuf, out_ref.at[pl.ds(tr * 8, 8), pl.ds(h * _CW, _CW)], sem)
            if scaled_ref is not None:
                pltpu.async_copy(sbuf, scaled_ref.at[pl.ds(tr * 8, 8), pl.ds(h * _CW, _CW)], sem)

        def drain_out(tr, h, obuf, sbuf, sem,
                      out_ref=out_ref, scaled_ref=scaled_ref):
            pltpu.make_async_copy(obuf, out_ref.at[pl.ds(tr * 8, 8), pl.ds(h * _CW, _CW)], sem).wait()
            if scaled_ref is not None:
                pltpu.make_async_copy(sbuf, scaled_ref.at[pl.ds(tr * 8, 8), pl.ds(h * _CW, _CW)], sem).wait()

        issue_in(base_tr, 0, bufs0, sem_a)

        def pair_body(t, _):
            tr = base_tr + t
            issue_in(tr, 1, bufs1, sem_b)
            wait_in(tr, 0, bufs0, sem_a)

            @pl.when(t > 0)
            def _():
                drain_out(tr, 0, ob0, sb0, sem_oa)

            _compute_chunk(bufs0, ob0, sb0, w0, w1, w2, w3, ws)
            issue_out(tr, 0, ob0, sb0, sem_oa)

            @pl.when(t < _TR_PER_W - 1)
            def _():
                issue_in(tr + 1, 0, bufs0, sem_a)

            wait_in(tr, 1, bufs1, sem_b)

            @pl.when(t > 0)
            def _():
                drain_out(tr, 1, ob1, sb1, sem_ob)

            _compute_chunk(bufs1, ob1, sb1, w0, w1, w2, w3, ws)
            issue_out(tr, 1, ob1, sb1, sem_ob)
            return 0

        lax.fori_loop(0, _TR_PER_W, pair_body, 0)
        drain_out(base_tr + _TR_PER_W - 1, 0, ob0, sb0, sem_oa)
        drain_out(base_tr + _TR_PER_W - 1, 1, ob1, sb1, sem_ob)


def _sc_call(weights, params):
    wb = jnp.broadcast_to(weights.reshape(20, 1), (20, 16))
    mesh = plsc.VectorSubcoreMesh(core_axis_name="c", subcore_axis_name="s",
                                  num_cores=_NC, num_subcores=_NS)
    buf = pltpu.VMEM((8, _CW), jnp.float32)
    f = pl.kernel(
        _sc_body,
        out_type=[jax.ShapeDtypeStruct((_ROWS, _COLS), jnp.float32)] * 6,
        mesh=mesh,
        compiler_params=pltpu.CompilerParams(use_tc_tiling_on_sc=True),
        scratch_types=[pltpu.VMEM((20, 16), jnp.float32)] + [buf] * 12 + [
            pltpu.SemaphoreType.DMA,
            pltpu.SemaphoreType.DMA,
            pltpu.SemaphoreType.DMA,
            pltpu.SemaphoreType.DMA,
        ],
    )
    return f(wb, *params)


def kernel(loss, prev_loss, weights, param_0, param_1, param_2, param_3,
           param_4, param_5, param_6, param_7, param_8, param_9, param_10,
           param_11, param_12, param_13, param_14, param_15):
    del loss, prev_loss
    params = (param_0, param_1, param_2, param_3, param_4, param_5, param_6,
              param_7, param_8, param_9, param_10, param_11, param_12,
              param_13, param_14, param_15)
    return tuple(_sc_call(weights, params))


# speedup vs baseline: 6.1615x; 1.1501x over previous
import jax
import jax.numpy as jnp
from jax import lax
from jax.experimental import pallas as pl
from jax.experimental.pallas import tpu as pltpu
from jax.experimental.pallas import tpu_sc as plsc

_ROWS = 2048
_COLS = 2048

_NC = 2
_NS = 16
_NW = _NC * _NS

_TR_PER_W = (_ROWS // 8) // _NW
_CW = _COLS // 2

_SC_GROUPS = (
    ((2, 6, 10, 14), (4, 8, 12, 16), None),
    ((3, 7, 11, 15), (5, 9, 13, 17), None),
)


def _compute_chunk(bufs, ob, sb, w0, w1, w2, w3, ws):
    @plsc.parallel_loop(0, 8 * (_CW // 16), unroll=8)
    def _(t):
        i = t >> 6
        ds = pl.ds((t & 63) * 16, 16)
        b0, b1, b2, b3 = bufs
        s = (b0[i, ds] * w0 + b1[i, ds] * w1) + (b2[i, ds] * w2 + b3[i, ds] * w3)
        ob[i, ds] = s
        if ws is not None:
            sb[i, ds] = s * ws


def _sc_body(wb, *refs):
    p = refs[0:8]
    o = refs[8:10]
    (wv, b00, b01, b02, b03, b10, b11, b12, b13, ob0, ob1, sb0, sb1,
     sem_a, sem_b, sem_oa, sem_ob) = refs[10:]
    bufs0 = (b00, b01, b02, b03)
    bufs1 = (b10, b11, b12, b13)
    wid = lax.axis_index("s") * _NC + lax.axis_index("c")
    base_tr = wid * _TR_PER_W
    pltpu.sync_copy(wb, wv)

    for g, (pidx_all, widx, sidx) in enumerate(_SC_GROUPS):
        pidx = tuple(range(4 * g, 4 * g + 4))
        w0, w1, w2, w3 = (wv[i] for i in widx)
        ws = wv[sidx] if sidx is not None else None
        out_ref = o[g]
        scaled_ref = None

        def in_slice(k, tr, h, pidx=pidx):
            return p[pidx[k]].at[pl.ds(tr * 8, 8), pl.ds(h * _CW, _CW)]

        def issue_in(tr, h, bufs, sem, pidx=pidx):
            for k in range(4):
                pltpu.async_copy(in_slice(k, tr, h, pidx), bufs[k], sem)

        def wait_in(tr, h, bufs, sem, pidx=pidx):
            for k in range(4):
                pltpu.make_async_copy(in_slice(k, tr, h, pidx), bufs[k], sem).wait()

        def issue_out(tr, h, obuf, sbuf, sem, out_ref=out_ref, scaled_ref=scaled_ref):
            pltpu.async_copy(obuf, out_ref.at[pl.ds(tr * 8, 8), pl.ds(h * _CW, _CW)], sem)
            if scaled_ref is not None:
                pltpu.async_copy(sbuf, scaled_ref.at[pl.ds(tr * 8, 8), pl.ds(h * _CW, _CW)], sem)

        def drain_out(tr, h, obuf, sbuf, sem, out_ref=out_ref, scaled_ref=scaled_ref):
            pltpu.make_async_copy(obuf, out_ref.at[pl.ds(tr * 8, 8), pl.ds(h * _CW, _CW)], sem).wait()
            if scaled_ref is not None:
                pltpu.make_async_copy(sbuf, scaled_ref.at[pl.ds(tr * 8, 8), pl.ds(h * _CW, _CW)], sem).wait()

        issue_in(base_tr, 0, bufs0, sem_a)

        def pair_body(t, _):
            tr = base_tr + t
            issue_in(tr, 1, bufs1, sem_b)
            wait_in(tr, 0, bufs0, sem_a)

            @pl.when(t > 0)
            def _():
                drain_out(tr, 0, ob0, sb0, sem_oa)

            _compute_chunk(bufs0, ob0, sb0, w0, w1, w2, w3, ws)
            issue_out(tr, 0, ob0, sb0, sem_oa)

            @pl.when(t < _TR_PER_W - 1)
            def _():
                issue_in(tr + 1, 0, bufs0, sem_a)

            wait_in(tr, 1, bufs1, sem_b)

            @pl.when(t > 0)
            def _():
                drain_out(tr, 1, ob1, sb1, sem_ob)

            _compute_chunk(bufs1, ob1, sb1, w0, w1, w2, w3, ws)
            issue_out(tr, 1, ob1, sb1, sem_ob)
            return 0

        lax.fori_loop(0, _TR_PER_W, pair_body, 0)
        drain_out(base_tr + _TR_PER_W - 1, 0, ob0, sb0, sem_oa)
        drain_out(base_tr + _TR_PER_W - 1, 1, ob1, sb1, sem_ob)


def _sc_call(weights, params8):
    wb = jnp.broadcast_to(weights.reshape(20, 1), (20, 16))
    mesh = plsc.VectorSubcoreMesh(core_axis_name="c", subcore_axis_name="s",
                                  num_cores=_NC, num_subcores=_NS)
    buf = pltpu.VMEM((8, _CW), jnp.float32)
    f = pl.kernel(
        _sc_body,
        out_type=[jax.ShapeDtypeStruct((_ROWS, _COLS), jnp.float32)] * 2,
        mesh=mesh,
        compiler_params=pltpu.CompilerParams(use_tc_tiling_on_sc=True),
        scratch_types=[pltpu.VMEM((20, 16), jnp.float32)] + [buf] * 12 + [
            pltpu.SemaphoreType.DMA,
            pltpu.SemaphoreType.DMA,
            pltpu.SemaphoreType.DMA,
            pltpu.SemaphoreType.DMA,
        ],
    )
    return f(wb, *params8)


_TC_BLK = 64


def _tc_body(w_ref, p0, p4, p8, p12, p1, p5, p9, p13, o18, o19, o22, o23):
    a = p0[...] * w_ref[2] + p4[...] * w_ref[6] + p8[...] * w_ref[10] + p12[...] * w_ref[14]
    b = p1[...] * w_ref[3] + p5[...] * w_ref[7] + p9[...] * w_ref[11] + p13[...] * w_ref[15]
    o18[...] = a
    o19[...] = b
    o22[...] = a * w_ref[18]
    o23[...] = b * w_ref[19]


def _tc_call(weights, params8):
    blk = pl.BlockSpec((_TC_BLK, _COLS), lambda i: (i, 0))
    return pl.pallas_call(
        _tc_body,
        grid=(_ROWS // _TC_BLK,),
        in_specs=[pl.BlockSpec(memory_space=pltpu.SMEM)] + [blk] * 8,
        out_specs=[blk] * 4,
        out_shape=[jax.ShapeDtypeStruct((_ROWS, _COLS), jnp.float32)] * 4,
    )(weights, *params8)


def kernel(loss, prev_loss, weights, param_0, param_1, param_2, param_3,
           param_4, param_5, param_6, param_7, param_8, param_9, param_10,
           param_11, param_12, param_13, param_14, param_15):
    del loss, prev_loss
    out20, out21 = _sc_call(
        weights,
        (param_2, param_6, param_10, param_14, param_3, param_7, param_11, param_15),
    )
    out18, out19, out22, out23 = _tc_call(
        weights,
        (param_0, param_4, param_8, param_12, param_1, param_5, param_9, param_13),
    )
    return (out18, out19, out20, out21, out22, out23)


# speedup vs baseline: 6.3518x; 1.0309x over previous
import jax
import jax.numpy as jnp
from jax import lax
from jax.experimental import pallas as pl
from jax.experimental.pallas import tpu as pltpu
from jax.experimental.pallas import tpu_sc as plsc

_ROWS = 2048
_COLS = 2048

_NC = 2
_NS = 16
_NW = _NC * _NS

_TR_PER_W = (_ROWS // 8) // _NW
_CW = _COLS // 2

_SC_GROUPS = (
    ((3, 7, 11, 15), (5, 9, 13, 17), None),
)


def _compute_chunk(bufs, ob, sb, w0, w1, w2, w3, ws):
    @plsc.parallel_loop(0, 8 * (_CW // 16), unroll=8)
    def _(t):
        i = t >> 6
        ds = pl.ds((t & 63) * 16, 16)
        b0, b1, b2, b3 = bufs
        s = (b0[i, ds] * w0 + b1[i, ds] * w1) + (b2[i, ds] * w2 + b3[i, ds] * w3)
        ob[i, ds] = s
        if ws is not None:
            sb[i, ds] = s * ws


def _sc_body(wb, *refs):
    np_in = 4 * len(_SC_GROUPS)
    p = refs[0:np_in]
    o = refs[np_in:np_in + len(_SC_GROUPS)]
    (wv, b00, b01, b02, b03, b10, b11, b12, b13, ob0, ob1, sb0, sb1,
     sem_a, sem_b, sem_oa, sem_ob) = refs[np_in + len(_SC_GROUPS):]
    bufs0 = (b00, b01, b02, b03)
    bufs1 = (b10, b11, b12, b13)
    wid = lax.axis_index("s") * _NC + lax.axis_index("c")
    base_tr = wid * _TR_PER_W
    pltpu.sync_copy(wb, wv)

    for g, (pidx_all, widx, sidx) in enumerate(_SC_GROUPS):
        pidx = tuple(range(4 * g, 4 * g + 4))
        w0, w1, w2, w3 = (wv[i] for i in widx)
        ws = wv[sidx] if sidx is not None else None
        out_ref = o[g]
        scaled_ref = None

        def in_slice(k, tr, h, pidx=pidx):
            return p[pidx[k]].at[pl.ds(tr * 8, 8), pl.ds(h * _CW, _CW)]

        def issue_in(tr, h, bufs, sem, pidx=pidx):
            for k in range(4):
                pltpu.async_copy(in_slice(k, tr, h, pidx), bufs[k], sem)

        def wait_in(tr, h, bufs, sem, pidx=pidx):
            for k in range(4):
                pltpu.make_async_copy(in_slice(k, tr, h, pidx), bufs[k], sem).wait()

        def issue_out(tr, h, obuf, sbuf, sem, out_ref=out_ref, scaled_ref=scaled_ref):
            pltpu.async_copy(obuf, out_ref.at[pl.ds(tr * 8, 8), pl.ds(h * _CW, _CW)], sem)
            if scaled_ref is not None:
                pltpu.async_copy(sbuf, scaled_ref.at[pl.ds(tr * 8, 8), pl.ds(h * _CW, _CW)], sem)

        def drain_out(tr, h, obuf, sbuf, sem, out_ref=out_ref, scaled_ref=scaled_ref):
            pltpu.make_async_copy(obuf, out_ref.at[pl.ds(tr * 8, 8), pl.ds(h * _CW, _CW)], sem).wait()
            if scaled_ref is not None:
                pltpu.make_async_copy(sbuf, scaled_ref.at[pl.ds(tr * 8, 8), pl.ds(h * _CW, _CW)], sem).wait()

        issue_in(base_tr, 0, bufs0, sem_a)

        def pair_body(t, _):
            tr = base_tr + t
            issue_in(tr, 1, bufs1, sem_b)
            wait_in(tr, 0, bufs0, sem_a)

            @pl.when(t > 0)
            def _():
                drain_out(tr, 0, ob0, sb0, sem_oa)

            _compute_chunk(bufs0, ob0, sb0, w0, w1, w2, w3, ws)
            issue_out(tr, 0, ob0, sb0, sem_oa)

            @pl.when(t < _TR_PER_W - 1)
            def _():
                issue_in(tr + 1, 0, bufs0, sem_a)

            wait_in(tr, 1, bufs1, sem_b)

            @pl.when(t > 0)
            def _():
                drain_out(tr, 1, ob1, sb1, sem_ob)

            _compute_chunk(bufs1, ob1, sb1, w0, w1, w2, w3, ws)
            issue_out(tr, 1, ob1, sb1, sem_ob)
            return 0

        lax.fori_loop(0, _TR_PER_W, pair_body, 0)
        drain_out(base_tr + _TR_PER_W - 1, 0, ob0, sb0, sem_oa)
        drain_out(base_tr + _TR_PER_W - 1, 1, ob1, sb1, sem_ob)


def _sc_call(weights, params8):
    wb = jnp.broadcast_to(weights.reshape(20, 1), (20, 16))
    mesh = plsc.VectorSubcoreMesh(core_axis_name="c", subcore_axis_name="s",
                                  num_cores=_NC, num_subcores=_NS)
    buf = pltpu.VMEM((8, _CW), jnp.float32)
    f = pl.kernel(
        _sc_body,
        out_type=[jax.ShapeDtypeStruct((_ROWS, _COLS), jnp.float32)] * len(_SC_GROUPS),
        mesh=mesh,
        compiler_params=pltpu.CompilerParams(use_tc_tiling_on_sc=True),
        scratch_types=[pltpu.VMEM((20, 16), jnp.float32)] + [buf] * 12 + [
            pltpu.SemaphoreType.DMA,
            pltpu.SemaphoreType.DMA,
            pltpu.SemaphoreType.DMA,
            pltpu.SemaphoreType.DMA,
        ],
    )
    return f(wb, *params8)


_TC_BLK = 64


def _tc_body(w_ref, p0, p4, p8, p12, p1, p5, p9, p13, p2, p6, p10, p14,
             o18, o19, o20, o22, o23):
    a = p0[...] * w_ref[2] + p4[...] * w_ref[6] + p8[...] * w_ref[10] + p12[...] * w_ref[14]
    b = p1[...] * w_ref[3] + p5[...] * w_ref[7] + p9[...] * w_ref[11] + p13[...] * w_ref[15]
    c = p2[...] * w_ref[4] + p6[...] * w_ref[8] + p10[...] * w_ref[12] + p14[...] * w_ref[16]
    o18[...] = a
    o19[...] = b
    o20[...] = c
    o22[...] = a * w_ref[18]
    o23[...] = b * w_ref[19]


def _tc_call(weights, params12):
    blk = pl.BlockSpec((_TC_BLK, _COLS), lambda i: (i, 0))
    return pl.pallas_call(
        _tc_body,
        grid=(_ROWS // _TC_BLK,),
        in_specs=[pl.BlockSpec(memory_space=pltpu.SMEM)] + [blk] * 12,
        out_specs=[blk] * 5,
        out_shape=[jax.ShapeDtypeStruct((_ROWS, _COLS), jnp.float32)] * 5,
    )(weights, *params12)


def kernel(loss, prev_loss, weights, param_0, param_1, param_2, param_3,
           param_4, param_5, param_6, param_7, param_8, param_9, param_10,
           param_11, param_12, param_13, param_14, param_15):
    del loss, prev_loss
    (out21,) = _sc_call(
        weights,
        (param_3, param_7, param_11, param_15),
    )
    out18, out19, out20, out22, out23 = _tc_call(
        weights,
        (param_0, param_4, param_8, param_12, param_1, param_5, param_9,
         param_13, param_2, param_6, param_10, param_14),
    )
    return (out18, out19, out20, out21, out22, out23)
